# baseline (speedup 1.0000x reference)
import functools

import jax
import jax.numpy as jnp
from jax import lax
from jax.experimental import pallas as pl
from jax.experimental.pallas import tpu as pltpu
from jax.experimental.pallas import tpu_sc as plsc

M = 65536
D_IN = 2048
D = 512
CRITERION = 0.5

NC = 2
NS = 16
L = 16
NW = NC * NS
CHUNK = 32
NBUF = 4
NG = D // L

M_SC = 20480
TC_BLK = 2048


ENC_KBLK = 256


def _encode_body(sound_ref, w_ref, out_ref):
    k = pl.program_id(0)

    @pl.when(k == 0)
    def _():
        out_ref[...] = jnp.zeros_like(out_ref)

    out_ref[...] += jnp.dot(sound_ref[...], w_ref[...],
                            preferred_element_type=jnp.float32)


def _encode(sound, W_enc):
    return pl.pallas_call(
        _encode_body,
        grid=(D_IN // ENC_KBLK,),
        in_specs=[
            pl.BlockSpec((1, ENC_KBLK), lambda k: (0, k)),
            pl.BlockSpec((ENC_KBLK, D), lambda k: (k, 0)),
        ],
        out_specs=pl.BlockSpec((1, D), lambda k: (0, 0)),
        out_shape=jax.ShapeDtypeStruct((1, D), jnp.float32),
        compiler_params=pltpu.CompilerParams(
            dimension_semantics=("arbitrary",)),
    )(sound, W_enc)


_sc_mesh = plsc.VectorSubcoreMesh(core_axis_name="c", subcore_axis_name="s")


def _make_sc_scan(m_sc):
    rows_per_w = m_sc // NW
    nchunks = rows_per_w // CHUNK
    assert rows_per_w % CHUNK == 0 and nchunks % NBUF == 0

    @functools.partial(
        pl.kernel,
        mesh=_sc_mesh,
        compiler_params=pltpu.CompilerParams(needs_layout_passes=False),
        out_type=jax.ShapeDtypeStruct((NW, L), jnp.float32),
        scratch_types=[
            [pltpu.VMEM((CHUNK, D), jnp.float32) for _ in range(NBUF)],
            pltpu.VMEM((D,), jnp.float32),
            pltpu.VMEM((L,), jnp.float32),
            [pltpu.SemaphoreType.DMA for _ in range(NBUF)],
        ],
    )
    def sc_scan(rep_hbm, bank_hbm, out_hbm, bufs, repv, minbuf, sems):
        wid = lax.axis_index("s") * NC + lax.axis_index("c")
        base = wid * rows_per_w

        pltpu.sync_copy(rep_hbm, repv)
        rep_vs = [repv[pl.ds(g * L, L)] for g in range(NG)]

        def start(c, buf, sem):
            pltpu.make_async_copy(
                bank_hbm.at[pl.ds(base + c * CHUNK, CHUNK)], buf, sem).start()

        def wait(buf, sem):
            pltpu.make_async_copy(
                bank_hbm.at[pl.ds(base, CHUNK)], buf, sem).wait()

        def scan_chunk(buf, m):
            def group_body(rg, m):
                r0 = rg * L
                for j in range(L):
                    acc = jnp.zeros((L,), jnp.float32)
                    for g in range(NG):
                        diff = buf[r0 + j, pl.ds(g * L, L)] - rep_vs[g]
                        acc = acc + diff * diff
                    m = jnp.minimum(m, plsc.cumsum(acc)[L - 1])
                return m
            return lax.fori_loop(0, CHUNK // L, group_body, m)

        for k in range(NBUF - 1):
            start(k, bufs[k], sems[k])

        def ring_body(p, m):
            c = NBUF * p
            for k in range(NBUF):
                nxt = c + k + (NBUF - 1)

                @pl.when(nxt < nchunks)
                def _(nxt=nxt, k=k):
                    start(nxt, bufs[(k + NBUF - 1) % NBUF],
                          sems[(k + NBUF - 1) % NBUF])

                wait(bufs[k], sems[k])
                m = scan_chunk(bufs[k], m)
            return m

        m = lax.fori_loop(0, nchunks // NBUF, ring_body, jnp.float32(jnp.inf))
        minbuf[...] = jnp.full((L,), m, jnp.float32)
        pltpu.sync_copy(minbuf, out_hbm.at[wid])

    return sc_scan


_sc_scan = _make_sc_scan(M_SC)


def _tc_scan_body(rep_ref, bank_ref, out_ref):
    i = pl.program_id(0)
    diff = bank_ref[...] - rep_ref[...]
    mn = jnp.min(jnp.sum(diff * diff, axis=1))

    @pl.when(i == 0)
    def _():
        out_ref[0, 0] = mn

    @pl.when(i > 0)
    def _():
        out_ref[0, 0] = jnp.minimum(out_ref[0, 0], mn)


def _tc_scan(rep, bank, row0, nrows):
    n_blk = nrows // TC_BLK
    assert nrows % TC_BLK == 0 and row0 % TC_BLK == 0
    blk0 = row0 // TC_BLK
    return pl.pallas_call(
        _tc_scan_body,
        grid=(n_blk,),
        in_specs=[
            pl.BlockSpec((1, D), lambda i: (0, 0)),
            pl.BlockSpec((TC_BLK, D), lambda i: (blk0 + i, 0)),
        ],
        out_specs=pl.BlockSpec(memory_space=pltpu.SMEM),
        out_shape=jax.ShapeDtypeStruct((1, 1), jnp.float32),
        compiler_params=pltpu.CompilerParams(
            dimension_semantics=("arbitrary",)),
    )(rep, bank)


M_TC_A = 34816


def kernel(sound, W_enc, memory_bank, ages):
    del ages
    rep = _encode(sound, W_enc)
    sc_mins = _sc_scan(rep.reshape(D), memory_bank)
    tc_min_a = _tc_scan(rep, memory_bank, M_SC, M_TC_A)
    sc_mins_b, bank_b = lax.optimization_barrier((sc_mins, memory_bank))
    tc_min_b = _tc_scan(rep, bank_b, M_SC + M_TC_A, M - M_SC - M_TC_A)
    min_sq = jnp.minimum(jnp.min(sc_mins_b),
                         jnp.minimum(tc_min_a[0, 0], tc_min_b[0, 0]))
    return (jnp.sqrt(min_sq) <= CRITERION).astype(jnp.float32).reshape(1)


# speedup vs baseline: 1.0670x; 1.0670x over previous
import functools

import jax
import jax.numpy as jnp
from jax import lax
from jax.experimental import pallas as pl
from jax.experimental.pallas import tpu as pltpu
from jax.experimental.pallas import tpu_sc as plsc

M = 65536
D_IN = 2048
D = 512
CRITERION = 0.5

NC = 2
NS = 16
L = 16
NW = NC * NS
CHUNK = 32
NBUF = 4
NG = D // L

M_SC = 24576
TC_BLK = 2048


def _encode_body(sound_ref, w_ref, out_ref):
    out_ref[...] = jnp.dot(sound_ref[...], w_ref[...],
                           preferred_element_type=jnp.float32)


def _encode(sound, W_enc):
    return pl.pallas_call(
        _encode_body,
        out_shape=jax.ShapeDtypeStruct((1, D), jnp.float32),
    )(sound, W_enc)


_sc_mesh = plsc.VectorSubcoreMesh(core_axis_name="c", subcore_axis_name="s")


def _make_sc_scan(m_sc):
    rows_per_w = m_sc // NW
    nchunks = rows_per_w // CHUNK
    assert rows_per_w % CHUNK == 0 and nchunks % NBUF == 0

    @functools.partial(
        pl.kernel,
        mesh=_sc_mesh,
        compiler_params=pltpu.CompilerParams(needs_layout_passes=False),
        out_type=jax.ShapeDtypeStruct((NW, L), jnp.float32),
        scratch_types=[
            [pltpu.VMEM((CHUNK, D), jnp.float32) for _ in range(NBUF)],
            pltpu.VMEM((D,), jnp.float32),
            pltpu.VMEM((L,), jnp.float32),
            [pltpu.SemaphoreType.DMA for _ in range(NBUF)],
        ],
    )
    def sc_scan(rep_hbm, bank_hbm, out_hbm, bufs, repv, minbuf, sems):
        wid = lax.axis_index("s") * NC + lax.axis_index("c")
        base = wid * rows_per_w

        pltpu.sync_copy(rep_hbm, repv)
        rep_vs = [repv[pl.ds(g * L, L)] for g in range(NG)]

        def start(c, buf, sem):
            pltpu.make_async_copy(
                bank_hbm.at[pl.ds(base + c * CHUNK, CHUNK)], buf, sem).start()

        def wait(buf, sem):
            pltpu.make_async_copy(
                bank_hbm.at[pl.ds(base, CHUNK)], buf, sem).wait()

        def scan_chunk(buf, m):
            def group_body(rg, m):
                r0 = rg * L
                for j in range(L):
                    acc = jnp.zeros((L,), jnp.float32)
                    for g in range(NG):
                        diff = buf[r0 + j, pl.ds(g * L, L)] - rep_vs[g]
                        acc = acc + diff * diff
                    m = jnp.minimum(m, plsc.cumsum(acc)[L - 1])
                return m
            return lax.fori_loop(0, CHUNK // L, group_body, m)

        for k in range(NBUF - 1):
            start(k, bufs[k], sems[k])

        def ring_body(p, m):
            c = NBUF * p
            for k in range(NBUF):
                nxt = c + k + (NBUF - 1)

                @pl.when(nxt < nchunks)
                def _(nxt=nxt, k=k):
                    start(nxt, bufs[(k + NBUF - 1) % NBUF],
                          sems[(k + NBUF - 1) % NBUF])

                wait(bufs[k], sems[k])
                m = scan_chunk(bufs[k], m)
            return m

        m = lax.fori_loop(0, nchunks // NBUF, ring_body, jnp.float32(jnp.inf))
        minbuf[...] = jnp.full((L,), m, jnp.float32)
        pltpu.sync_copy(minbuf, out_hbm.at[wid])

    return sc_scan


_sc_scan = _make_sc_scan(M_SC)


def _tc_scan_body(rep_ref, bank_ref, out_ref):
    i = pl.program_id(0)
    diff = bank_ref[...] - rep_ref[...]
    mn = jnp.min(jnp.sum(diff * diff, axis=1))

    @pl.when(i == 0)
    def _():
        out_ref[0, 0] = mn

    @pl.when(i > 0)
    def _():
        out_ref[0, 0] = jnp.minimum(out_ref[0, 0], mn)


def _tc_scan(rep, bank, row0, nrows):
    n_blk = nrows // TC_BLK
    assert nrows % TC_BLK == 0 and row0 % TC_BLK == 0
    blk0 = row0 // TC_BLK
    return pl.pallas_call(
        _tc_scan_body,
        grid=(n_blk,),
        in_specs=[
            pl.BlockSpec((1, D), lambda i: (0, 0)),
            pl.BlockSpec((TC_BLK, D), lambda i: (blk0 + i, 0)),
        ],
        out_specs=pl.BlockSpec(memory_space=pltpu.SMEM),
        out_shape=jax.ShapeDtypeStruct((1, 1), jnp.float32),
        compiler_params=pltpu.CompilerParams(
            dimension_semantics=("arbitrary",)),
    )(rep, bank)


def kernel(sound, W_enc, memory_bank, ages):
    del ages
    rep = _encode(sound, W_enc)
    sc_mins = _sc_scan(rep.reshape(D), memory_bank)
    tc_min = _tc_scan(rep, memory_bank, M_SC, M - M_SC)
    min_sq = jnp.minimum(jnp.min(sc_mins), tc_min[0, 0])
    return (jnp.sqrt(min_sq) <= CRITERION).astype(jnp.float32).reshape(1)


# speedup vs baseline: 1.0701x; 1.0029x over previous
import functools

import jax
import jax.numpy as jnp
from jax import lax
from jax.experimental import pallas as pl
from jax.experimental.pallas import tpu as pltpu
from jax.experimental.pallas import tpu_sc as plsc

M = 65536
D_IN = 2048
D = 512
CRITERION = 0.5

NC = 2
NS = 16
L = 16
NW = NC * NS
CHUNK = 32
NBUF = 4
NG = D // L

M_SC = 24576
TC_BLK = 2048


def _encode_body(sound_ref, w_ref, out_ref):
    out_ref[...] = jnp.dot(sound_ref[...], w_ref[...],
                           preferred_element_type=jnp.float32)


def _encode(sound, W_enc):
    return pl.pallas_call(
        _encode_body,
        out_shape=jax.ShapeDtypeStruct((1, D), jnp.float32),
    )(sound, W_enc)


_sc_mesh = plsc.VectorSubcoreMesh(core_axis_name="c", subcore_axis_name="s")


def _make_sc_scan(m_sc):
    rows_per_w = m_sc // NW
    nchunks = rows_per_w // CHUNK
    assert rows_per_w % CHUNK == 0 and nchunks % NBUF == 0

    @functools.partial(
        pl.kernel,
        mesh=_sc_mesh,
        compiler_params=pltpu.CompilerParams(needs_layout_passes=False,
                                             skip_device_barrier=True),
        out_type=jax.ShapeDtypeStruct((NW, L), jnp.float32),
        scratch_types=[
            [pltpu.VMEM((CHUNK, D), jnp.float32) for _ in range(NBUF)],
            pltpu.VMEM((D,), jnp.float32),
            pltpu.VMEM((L,), jnp.float32),
            [pltpu.SemaphoreType.DMA for _ in range(NBUF)],
        ],
    )
    def sc_scan(rep_hbm, bank_hbm, out_hbm, bufs, repv, minbuf, sems):
        wid = lax.axis_index("s") * NC + lax.axis_index("c")
        base = wid * rows_per_w

        pltpu.sync_copy(rep_hbm, repv)
        rep_vs = [repv[pl.ds(g * L, L)] for g in range(NG)]

        def start(c, buf, sem):
            pltpu.make_async_copy(
                bank_hbm.at[pl.ds(base + c * CHUNK, CHUNK)], buf, sem).start()

        def wait(buf, sem):
            pltpu.make_async_copy(
                bank_hbm.at[pl.ds(base, CHUNK)], buf, sem).wait()

        def scan_chunk(buf, m):
            def group_body(rg, m):
                r0 = rg * L
                for j in range(L):
                    acc = jnp.zeros((L,), jnp.float32)
                    for g in range(NG):
                        diff = buf[r0 + j, pl.ds(g * L, L)] - rep_vs[g]
                        acc = acc + diff * diff
                    m = jnp.minimum(m, plsc.cumsum(acc)[L - 1])
                return m
            return lax.fori_loop(0, CHUNK // L, group_body, m)

        for k in range(NBUF - 1):
            start(k, bufs[k], sems[k])

        def ring_body(p, m):
            c = NBUF * p
            for k in range(NBUF):
                nxt = c + k + (NBUF - 1)

                @pl.when(nxt < nchunks)
                def _(nxt=nxt, k=k):
                    start(nxt, bufs[(k + NBUF - 1) % NBUF],
                          sems[(k + NBUF - 1) % NBUF])

                wait(bufs[k], sems[k])
                m = scan_chunk(bufs[k], m)
            return m

        m = lax.fori_loop(0, nchunks // NBUF, ring_body, jnp.float32(jnp.inf))
        minbuf[...] = jnp.full((L,), m, jnp.float32)
        pltpu.sync_copy(minbuf, out_hbm.at[wid])

    return sc_scan


_sc_scan = _make_sc_scan(M_SC)


def _tc_scan_body(rep_ref, bank_ref, out_ref):
    i = pl.program_id(0)
    diff = bank_ref[...] - rep_ref[...]
    mn = jnp.min(jnp.sum(diff * diff, axis=1))

    @pl.when(i == 0)
    def _():
        out_ref[0, 0] = mn

    @pl.when(i > 0)
    def _():
        out_ref[0, 0] = jnp.minimum(out_ref[0, 0], mn)


def _tc_scan(rep, bank, row0, nrows):
    n_blk = nrows // TC_BLK
    assert nrows % TC_BLK == 0 and row0 % TC_BLK == 0
    blk0 = row0 // TC_BLK
    return pl.pallas_call(
        _tc_scan_body,
        grid=(n_blk,),
        in_specs=[
            pl.BlockSpec((1, D), lambda i: (0, 0)),
            pl.BlockSpec((TC_BLK, D), lambda i: (blk0 + i, 0)),
        ],
        out_specs=pl.BlockSpec(memory_space=pltpu.SMEM),
        out_shape=jax.ShapeDtypeStruct((1, 1), jnp.float32),
        compiler_params=pltpu.CompilerParams(
            dimension_semantics=("arbitrary",)),
    )(rep, bank)


def kernel(sound, W_enc, memory_bank, ages):
    del ages
    rep = _encode(sound, W_enc)
    sc_mins = _sc_scan(rep.reshape(D), memory_bank)
    tc_min = _tc_scan(rep, memory_bank, M_SC, M - M_SC)
    min_sq = jnp.minimum(jnp.min(sc_mins), tc_min[0, 0])
    return (jnp.sqrt(min_sq) <= CRITERION).astype(jnp.float32).reshape(1)


# speedup vs baseline: 1.0716x; 1.0014x over previous
import functools

import jax
import jax.numpy as jnp
from jax import lax
from jax.experimental import pallas as pl
from jax.experimental.pallas import tpu as pltpu
from jax.experimental.pallas import tpu_sc as plsc

M = 65536
D_IN = 2048
D = 512
CRITERION = 0.5

NC = 2
NS = 16
L = 16
NW = NC * NS
CHUNK = 32
NBUF = 4
NG = D // L

M_SC = 24576
TC_BLK = 2048


def _encode_body(sound_ref, w_ref, out_ref):
    out_ref[...] = jnp.dot(sound_ref[...], w_ref[...],
                           preferred_element_type=jnp.float32)


def _encode(sound, W_enc):
    return pl.pallas_call(
        _encode_body,
        out_shape=jax.ShapeDtypeStruct((1, D), jnp.float32),
    )(sound, W_enc)


_sc_mesh = plsc.VectorSubcoreMesh(core_axis_name="c", subcore_axis_name="s")


def _make_sc_scan(m_sc):
    rows_per_w = m_sc // NW
    nchunks = rows_per_w // CHUNK
    assert rows_per_w % CHUNK == 0 and nchunks % NBUF == 0

    @functools.partial(
        pl.kernel,
        mesh=_sc_mesh,
        compiler_params=pltpu.CompilerParams(needs_layout_passes=False),
        out_type=jax.ShapeDtypeStruct((NW * L,), jnp.float32),
        scratch_types=[
            [pltpu.VMEM((CHUNK, D), jnp.float32) for _ in range(NBUF)],
            pltpu.VMEM((D,), jnp.float32),
            pltpu.VMEM((L,), jnp.float32),
            [pltpu.SemaphoreType.DMA for _ in range(NBUF)],
        ],
    )
    def sc_scan(rep_hbm, bank_hbm, out_hbm, bufs, repv, minbuf, sems):
        wid = lax.axis_index("s") * NC + lax.axis_index("c")
        base = wid * rows_per_w

        pltpu.sync_copy(rep_hbm, repv)
        rep_vs = [repv[pl.ds(g * L, L)] for g in range(NG)]

        def start(c, buf, sem):
            pltpu.make_async_copy(
                bank_hbm.at[pl.ds(base + c * CHUNK, CHUNK)], buf, sem).start()

        def wait(buf, sem):
            pltpu.make_async_copy(
                bank_hbm.at[pl.ds(base, CHUNK)], buf, sem).wait()

        def scan_chunk(buf, m):
            def group_body(rg, m):
                r0 = rg * L
                for j in range(L):
                    acc = jnp.zeros((L,), jnp.float32)
                    for g in range(NG):
                        diff = buf[r0 + j, pl.ds(g * L, L)] - rep_vs[g]
                        acc = acc + diff * diff
                    m = jnp.minimum(m, plsc.cumsum(acc)[L - 1])
                return m
            return lax.fori_loop(0, CHUNK // L, group_body, m)

        for k in range(NBUF - 1):
            start(k, bufs[k], sems[k])

        def ring_body(p, m):
            c = NBUF * p
            for k in range(NBUF):
                nxt = c + k + (NBUF - 1)

                @pl.when(nxt < nchunks)
                def _(nxt=nxt, k=k):
                    start(nxt, bufs[(k + NBUF - 1) % NBUF],
                          sems[(k + NBUF - 1) % NBUF])

                wait(bufs[k], sems[k])
                m = scan_chunk(bufs[k], m)
            return m

        m = lax.fori_loop(0, nchunks // NBUF, ring_body, jnp.float32(jnp.inf))
        minbuf[...] = jnp.full((L,), m, jnp.float32)
        pltpu.sync_copy(minbuf, out_hbm.at[pl.ds(wid * L, L)])

    return sc_scan


_sc_scan = _make_sc_scan(M_SC)


def _tc_scan_body(rep_ref, bank_ref, out_ref):
    i = pl.program_id(0)
    diff = bank_ref[...] - rep_ref[...]
    mn = jnp.min(jnp.sum(diff * diff, axis=1))

    @pl.when(i == 0)
    def _():
        out_ref[0, 0] = mn

    @pl.when(i > 0)
    def _():
        out_ref[0, 0] = jnp.minimum(out_ref[0, 0], mn)


def _tc_scan(rep, bank, row0, nrows):
    n_blk = nrows // TC_BLK
    assert nrows % TC_BLK == 0 and row0 % TC_BLK == 0
    blk0 = row0 // TC_BLK
    return pl.pallas_call(
        _tc_scan_body,
        grid=(n_blk,),
        in_specs=[
            pl.BlockSpec((1, D), lambda i: (0, 0)),
            pl.BlockSpec((TC_BLK, D), lambda i: (blk0 + i, 0)),
        ],
        out_specs=pl.BlockSpec(memory_space=pltpu.SMEM),
        out_shape=jax.ShapeDtypeStruct((1, 1), jnp.float32),
        compiler_params=pltpu.CompilerParams(
            dimension_semantics=("arbitrary",)),
    )(rep, bank)


def kernel(sound, W_enc, memory_bank, ages):
    del ages
    rep = _encode(sound, W_enc)
    sc_mins = _sc_scan(rep.reshape(D), memory_bank)
    tc_min = _tc_scan(rep, memory_bank, M_SC, M - M_SC)
    min_sq = jnp.minimum(jnp.min(sc_mins), tc_min[0, 0])
    return (jnp.sqrt(min_sq) <= CRITERION).astype(jnp.float32).reshape(1)
